# baseline (speedup 1.0000x reference)
import dataclasses
import functools

import jax
import jax.numpy as jnp
from jax import lax
from jax.experimental import pallas as pl
from jax.experimental.pallas import tpu as pltpu
from jax.experimental.pallas import tpu_sc as plsc

N = 16384
D = 64
NC = 2
NS = 16
NW = NC * NS
Q = 128
NQ = 2
SB = Q * NQ
ROWS_PER_TILE = N // NS
NGB = 3
NIB = 4


def _sc_spmm(rows, cols, vals, weight):
    nnz = rows.shape[0]
    nsb = pl.cdiv(nnz, NW * SB)
    epw = nsb * SB
    pad = epw * NW - nnz
    if pad:
        rows = jnp.concatenate([rows, jnp.zeros((pad,), rows.dtype)])
        cols = jnp.concatenate([cols, jnp.zeros((pad,), cols.dtype)])
        vals = jnp.concatenate([vals, jnp.zeros((pad,), vals.dtype)])
    rows = rows.reshape(-1, Q)

    mesh = plsc.VectorSubcoreMesh(core_axis_name="c", subcore_axis_name="s")
    cp = pltpu.CompilerParams()
    if "needs_layout_passes" in pltpu.CompilerParams.__dataclass_fields__:
        cp = dataclasses.replace(cp, needs_layout_passes=False)
    if "use_tc_tiling_on_sc" in pltpu.CompilerParams.__dataclass_fields__:
        cp = dataclasses.replace(cp, use_tc_tiling_on_sc=False)

    scratch = (
        [pltpu.VMEM((NQ, Q), jnp.int32) for _ in range(NIB)]
        + [pltpu.VMEM((SB,), jnp.int32) for _ in range(NIB)]
        + [pltpu.VMEM((SB,), jnp.float32) for _ in range(NIB)]
        + [pltpu.VMEM((SB, D), jnp.float32) for _ in range(NGB)]
        + [pltpu.VMEM_SHARED((N, D), jnp.float32)]
        + [pltpu.SemaphoreType.DMA for _ in range(NIB)]
        + [pltpu.SemaphoreType.DMA for _ in range(NGB)]
        + [pltpu.SemaphoreType.DMA for _ in range(NGB)]
    )

    @functools.partial(
        pl.kernel,
        mesh=mesh,
        compiler_params=cp,
        out_type=jax.ShapeDtypeStruct((NC, N, D), jnp.float32),
        scratch_types=scratch,
    )
    def spmm(rows_hbm, cols_hbm, vals_hbm, w_hbm, part_hbm, *refs):
        rows_v = refs[0:NIB]
        cols_v = refs[NIB:2 * NIB]
        vals_s = refs[2 * NIB:3 * NIB]
        g_v = refs[3 * NIB:3 * NIB + NGB]
        acc = refs[3 * NIB + NGB]
        sem_i = refs[3 * NIB + NGB + 1:3 * NIB + NGB + 1 + NIB]
        sem_g = refs[3 * NIB + NGB + 1 + NIB:3 * NIB + NGB + 1 + NIB + NGB]
        sem_s = refs[3 * NIB + NGB + 1 + NIB + NGB:]

        cid = lax.axis_index("c")
        sid = lax.axis_index("s")

        @pl.loop(0, SB)
        def _(r):
            for j in range(D // 16):
                g_v[0][r, pl.ds(j * 16, 16)] = jnp.zeros((16,), jnp.float32)

        zbase = sid * ROWS_PER_TILE
        for z in range(ROWS_PER_TILE // SB):
            pltpu.sync_copy(g_v[0], acc.at[pl.ds(zbase + z * SB, SB)])
        plsc.subcore_barrier()

        wid = sid * NC + cid
        base = wid * epw

        def issue_idx(s):
            b = s % NIB
            off = base + s * SB
            return [
                pltpu.async_copy(rows_hbm.at[pl.ds(off // Q, NQ)], rows_v[b],
                                 sem_i[b]),
                pltpu.async_copy(cols_hbm.at[pl.ds(off, SB)], cols_v[b],
                                 sem_i[b]),
                pltpu.async_copy(vals_hbm.at[pl.ds(off, SB)], vals_s[b],
                                 sem_i[b]),
            ]

        def issue_gather(s):
            b = s % NGB
            return pltpu.async_copy(w_hbm.at[cols_v[s % NIB]], g_v[b],
                                    sem_g[b])

        idx_c = {0: issue_idx(0)}
        if nsb > 1:
            idx_c[1] = issue_idx(1)
        for c in idx_c.pop(0):
            c.wait()
        g_c = {0: issue_gather(0)}
        s_c = {}

        for s in range(nsb):
            b = s % NGB
            if s - 2 >= 0:
                for c in s_c.pop(s - 2):
                    c.wait()
            if s + 1 < nsb:
                for c in idx_c.pop(s + 1):
                    c.wait()
                g_c[s + 1] = issue_gather(s + 1)
            g_c.pop(s).wait()

            vref = vals_s[s % NIB]
            gref = g_v[b]

            @pl.loop(0, SB, step=16)
            def _(e0):
                v = vref[pl.ds(e0, 16)]
                for u in range(16):
                    vb = jnp.full((16,), v[u], jnp.float32)
                    for j in range(D // 16):
                        gref[e0 + u, pl.ds(j * 16, 16)] = (
                            gref[e0 + u, pl.ds(j * 16, 16)] * vb)

            s_c[s] = [
                pltpu.async_copy(gref.at[pl.ds(q * Q, Q)],
                                 acc.at[rows_v[s % NIB].at[q]], sem_s[b],
                                 add=True)
                for q in range(NQ)
            ]
            if s + 2 < nsb:
                idx_c[s + 2] = issue_idx(s + 2)

        for cs in s_c.values():
            for c in cs:
                c.wait()

        plsc.subcore_barrier()
        pltpu.sync_copy(
            acc.at[pl.ds(sid * ROWS_PER_TILE, ROWS_PER_TILE)],
            part_hbm.at[cid, pl.ds(sid * ROWS_PER_TILE, ROWS_PER_TILE)],
        )

    return spmm(rows, cols, vals, weight)


def _tc_combine(part):
    def body(p_ref, o_ref):
        o_ref[...] = p_ref[0] + p_ref[1]

    BR = 512
    return pl.pallas_call(
        body,
        out_shape=jax.ShapeDtypeStruct((N, D), jnp.float32),
        grid=(N // BR,),
        in_specs=[pl.BlockSpec((NC, BR, D), lambda i: (0, i, 0))],
        out_specs=pl.BlockSpec((BR, D), lambda i: (i, 0)),
    )(part)


def kernel(rows, cols, vals, weight):
    rows = rows.astype(jnp.int32)
    cols = cols.astype(jnp.int32)
    part = _sc_spmm(rows, cols, vals, weight)
    return _tc_combine(part)


# speedup vs baseline: 1.2873x; 1.2873x over previous
import dataclasses
import functools

import jax
import jax.numpy as jnp
from jax import lax
from jax.experimental import pallas as pl
from jax.experimental.pallas import tpu as pltpu
from jax.experimental.pallas import tpu_sc as plsc

N = 16384
D = 64
NC = 2
NS = 16
NW = NC * NS
Q = 128
NQ = 2
SB = Q * NQ
ROWS_PER_TILE = N // NS
NGB = 3
NIB = 4


def _sc_spmm(rows, cols, vals, weight):
    nnz = rows.shape[0]
    nsb = pl.cdiv(nnz, NW * SB)
    epw = nsb * SB
    pad = epw * NW - nnz
    if pad:
        rows = jnp.concatenate([rows, jnp.zeros((pad,), rows.dtype)])
        cols = jnp.concatenate([cols, jnp.zeros((pad,), cols.dtype)])
        vals = jnp.concatenate([vals, jnp.zeros((pad,), vals.dtype)])
    rows = rows.reshape(-1, Q)

    mesh = plsc.VectorSubcoreMesh(core_axis_name="c", subcore_axis_name="s")
    cp = pltpu.CompilerParams()
    if "needs_layout_passes" in pltpu.CompilerParams.__dataclass_fields__:
        cp = dataclasses.replace(cp, needs_layout_passes=False)
    if "use_tc_tiling_on_sc" in pltpu.CompilerParams.__dataclass_fields__:
        cp = dataclasses.replace(cp, use_tc_tiling_on_sc=False)

    scratch = (
        [pltpu.VMEM((NQ, Q), jnp.int32) for _ in range(NIB)]
        + [pltpu.VMEM((SB,), jnp.int32) for _ in range(NIB)]
        + [pltpu.VMEM((SB,), jnp.float32) for _ in range(NIB)]
        + [pltpu.VMEM((SB, D), jnp.float32) for _ in range(NGB)]
        + [pltpu.VMEM_SHARED((N, D), jnp.float32)]
        + [pltpu.SemaphoreType.DMA for _ in range(NIB)]
        + [pltpu.SemaphoreType.DMA for _ in range(NGB)]
        + [pltpu.SemaphoreType.DMA for _ in range(NGB)]
    )

    @functools.partial(
        pl.kernel,
        mesh=mesh,
        compiler_params=cp,
        out_type=jax.ShapeDtypeStruct((NC, N, D), jnp.float32),
        scratch_types=scratch,
    )
    def spmm(rows_hbm, cols_hbm, vals_hbm, w_hbm, part_hbm, *refs):
        rows_v = refs[0:NIB]
        cols_v = refs[NIB:2 * NIB]
        vals_s = refs[2 * NIB:3 * NIB]
        g_v = refs[3 * NIB:3 * NIB + NGB]
        acc = refs[3 * NIB + NGB]
        sem_i = refs[3 * NIB + NGB + 1:3 * NIB + NGB + 1 + NIB]
        sem_g = refs[3 * NIB + NGB + 1 + NIB:3 * NIB + NGB + 1 + NIB + NGB]
        sem_s = refs[3 * NIB + NGB + 1 + NIB + NGB:]

        cid = lax.axis_index("c")
        sid = lax.axis_index("s")

        @pl.loop(0, SB)
        def _(r):
            for j in range(D // 16):
                g_v[0][r, pl.ds(j * 16, 16)] = jnp.zeros((16,), jnp.float32)

        zbase = sid * ROWS_PER_TILE
        for z in range(ROWS_PER_TILE // SB):
            pltpu.sync_copy(g_v[0], acc.at[pl.ds(zbase + z * SB, SB)])
        plsc.subcore_barrier()

        wid = sid * NC + cid
        base = wid * epw

        def issue_idx(s):
            b = s % NIB
            off = base + s * SB
            return [
                pltpu.async_copy(rows_hbm.at[pl.ds(off // Q, NQ)], rows_v[b],
                                 sem_i[b]),
                pltpu.async_copy(cols_hbm.at[pl.ds(off, SB)], cols_v[b],
                                 sem_i[b]),
                pltpu.async_copy(vals_hbm.at[pl.ds(off, SB)], vals_s[b],
                                 sem_i[b]),
            ]

        def issue_gather(s):
            b = s % NGB
            return pltpu.async_copy(w_hbm.at[cols_v[s % NIB]], g_v[b],
                                    sem_g[b])

        idx_c = {0: issue_idx(0)}
        if nsb > 1:
            idx_c[1] = issue_idx(1)
        for c in idx_c.pop(0):
            c.wait()
        g_c = {0: issue_gather(0)}
        s_c = {}

        for s in range(nsb):
            b = s % NGB
            if s - 2 >= 0:
                for c in s_c.pop(s - 2):
                    c.wait()
            if s + 1 < nsb:
                for c in idx_c.pop(s + 1):
                    c.wait()
                g_c[s + 1] = issue_gather(s + 1)
            g_c.pop(s).wait()

            vref = vals_s[s % NIB]
            gref = g_v[b]

            @pl.loop(0, SB, step=4)
            def _(e0):
                for u in range(4):
                    e = e0 + u
                    vb = plsc.load_gather(vref,
                                          [jnp.full((16,), e, jnp.int32)])
                    for j in range(D // 16):
                        gref[e, pl.ds(j * 16, 16)] = (
                            gref[e, pl.ds(j * 16, 16)] * vb)

            s_c[s] = [
                pltpu.async_copy(gref.at[pl.ds(q * Q, Q)],
                                 acc.at[rows_v[s % NIB].at[q]], sem_s[b],
                                 add=True)
                for q in range(NQ)
            ]
            if s + 2 < nsb:
                idx_c[s + 2] = issue_idx(s + 2)

        for cs in s_c.values():
            for c in cs:
                c.wait()

        plsc.subcore_barrier()
        pltpu.sync_copy(
            acc.at[pl.ds(sid * ROWS_PER_TILE, ROWS_PER_TILE)],
            part_hbm.at[cid, pl.ds(sid * ROWS_PER_TILE, ROWS_PER_TILE)],
        )

    return spmm(rows, cols, vals, weight)


def _tc_combine(part):
    def body(p_ref, o_ref):
        o_ref[...] = p_ref[0] + p_ref[1]

    BR = 512
    return pl.pallas_call(
        body,
        out_shape=jax.ShapeDtypeStruct((N, D), jnp.float32),
        grid=(N // BR,),
        in_specs=[pl.BlockSpec((NC, BR, D), lambda i: (0, i, 0))],
        out_specs=pl.BlockSpec((BR, D), lambda i: (i, 0)),
    )(part)


def kernel(rows, cols, vals, weight):
    rows = rows.astype(jnp.int32)
    cols = cols.astype(jnp.int32)
    part = _sc_spmm(rows, cols, vals, weight)
    return _tc_combine(part)


# speedup vs baseline: 1.3553x; 1.0529x over previous
import dataclasses
import functools

import jax
import jax.numpy as jnp
from jax import lax
from jax.experimental import pallas as pl
from jax.experimental.pallas import tpu as pltpu
from jax.experimental.pallas import tpu_sc as plsc

N = 16384
D = 64
NC = 2
NS = 16
NW = NC * NS
Q = 96
NQ = 2
SB = Q * NQ
ROWS_PER_TILE = N // NS
NGB = 3


def _sc_spmm(rows, cols, vals, weight):
    nnz = rows.shape[0]
    nsb = pl.cdiv(nnz, NW * SB)
    epw = nsb * SB
    pad = epw * NW - nnz
    if pad:
        rows = jnp.concatenate([rows, jnp.zeros((pad,), rows.dtype)])
        cols = jnp.concatenate([cols, jnp.zeros((pad,), cols.dtype)])
        vals = jnp.concatenate([vals, jnp.zeros((pad,), vals.dtype)])
    rows = rows.reshape(-1, Q)

    mesh = plsc.VectorSubcoreMesh(core_axis_name="c", subcore_axis_name="s")
    cp = pltpu.CompilerParams()
    if "needs_layout_passes" in pltpu.CompilerParams.__dataclass_fields__:
        cp = dataclasses.replace(cp, needs_layout_passes=False)
    if "use_tc_tiling_on_sc" in pltpu.CompilerParams.__dataclass_fields__:
        cp = dataclasses.replace(cp, use_tc_tiling_on_sc=False)

    @functools.partial(
        pl.kernel,
        mesh=mesh,
        compiler_params=cp,
        out_type=jax.ShapeDtypeStruct((NC, N, D), jnp.float32),
        scratch_types=[
            pltpu.VMEM((NQ * nsb, Q), jnp.int32),
            pltpu.VMEM((epw,), jnp.int32),
            pltpu.VMEM((epw,), jnp.float32),
            pltpu.VMEM((NGB, SB, D), jnp.float32),
            pltpu.VMEM_SHARED((N, D), jnp.float32),
            pltpu.SemaphoreType.DMA,
            pltpu.SemaphoreType.DMA((NGB,)),
            pltpu.SemaphoreType.DMA((NGB,)),
        ],
    )
    def spmm(rows_hbm, cols_hbm, vals_hbm, w_hbm, part_hbm,
             rows_v, cols_v, vals_v, g_v, acc, sem_i, sem_g, sem_s):
        cid = lax.axis_index("c")
        sid = lax.axis_index("s")
        wid = sid * NC + cid
        base = wid * epw

        idx_c = [
            pltpu.async_copy(rows_hbm.at[pl.ds(wid * NQ * nsb, NQ * nsb)],
                             rows_v, sem_i),
            pltpu.async_copy(cols_hbm.at[pl.ds(base, epw)], cols_v, sem_i),
            pltpu.async_copy(vals_hbm.at[pl.ds(base, epw)], vals_v, sem_i),
        ]

        @pl.loop(0, SB)
        def _(r):
            for j in range(D // 16):
                g_v[0, r, pl.ds(j * 16, 16)] = jnp.zeros((16,), jnp.float32)

        zbase = sid * ROWS_PER_TILE
        done = 0
        while done < ROWS_PER_TILE:
            step = min(SB, ROWS_PER_TILE - done)
            pltpu.sync_copy(g_v.at[0, pl.ds(0, step)],
                            acc.at[pl.ds(zbase + done, step)])
            done += step
        plsc.subcore_barrier()

        for c in idx_c:
            c.wait()

        def issue_gather(s):
            b = s % NGB
            return pltpu.async_copy(w_hbm.at[cols_v.at[pl.ds(s * SB, SB)]],
                                    g_v.at[b], sem_g.at[b])

        g_c = {0: issue_gather(0)}
        s_c = {}

        for s in range(nsb):
            b = s % NGB
            if s - 2 >= 0:
                for c in s_c.pop(s - 2):
                    c.wait()
            if s + 1 < nsb:
                g_c[s + 1] = issue_gather(s + 1)
            g_c.pop(s).wait()

            @pl.loop(0, SB, step=4)
            def _(e0):
                for u in range(4):
                    e = e0 + u
                    vb = plsc.load_gather(
                        vals_v, [jnp.full((16,), s * SB + e, jnp.int32)])
                    for j in range(D // 16):
                        g_v[b, e, pl.ds(j * 16, 16)] = (
                            g_v[b, e, pl.ds(j * 16, 16)] * vb)

            s_c[s] = [
                pltpu.async_copy(g_v.at[b, pl.ds(q * Q, Q)],
                                 acc.at[rows_v.at[NQ * s + q]], sem_s.at[b],
                                 add=True)
                for q in range(NQ)
            ]

        for cs in s_c.values():
            for c in cs:
                c.wait()

        plsc.subcore_barrier()
        pltpu.sync_copy(
            acc.at[pl.ds(sid * ROWS_PER_TILE, ROWS_PER_TILE)],
            part_hbm.at[cid, pl.ds(sid * ROWS_PER_TILE, ROWS_PER_TILE)],
        )

    return spmm(rows, cols, vals, weight)


def _tc_combine(part):
    def body(p_ref, o_ref):
        o_ref[...] = p_ref[0] + p_ref[1]

    BR = 512
    return pl.pallas_call(
        body,
        out_shape=jax.ShapeDtypeStruct((N, D), jnp.float32),
        grid=(N // BR,),
        in_specs=[pl.BlockSpec((NC, BR, D), lambda i: (0, i, 0))],
        out_specs=pl.BlockSpec((BR, D), lambda i: (i, 0)),
    )(part)


def kernel(rows, cols, vals, weight):
    rows = rows.astype(jnp.int32)
    cols = cols.astype(jnp.int32)
    part = _sc_spmm(rows, cols, vals, weight)
    return _tc_combine(part)


# speedup vs baseline: 1.6386x; 1.2090x over previous
import dataclasses
import functools

import jax
import jax.numpy as jnp
from jax import lax
from jax.experimental import pallas as pl
from jax.experimental.pallas import tpu as pltpu
from jax.experimental.pallas import tpu_sc as plsc

N = 16384
D = 64
NC = 2
NS = 16
NW = NC * NS
Q = 112
NQ = 2
SB = Q * NQ
ROWS_PER_TILE = N // NS
NGB = 3
NCB = 4
NSB0 = 29
NSB1 = 46
NSBM = max(NSB0, NSB1)
EPWM = NSBM * SB


def _mk_cp():
    cp = pltpu.CompilerParams()
    if "needs_layout_passes" in pltpu.CompilerParams.__dataclass_fields__:
        cp = dataclasses.replace(cp, needs_layout_passes=False)
    if "use_tc_tiling_on_sc" in pltpu.CompilerParams.__dataclass_fields__:
        cp = dataclasses.replace(cp, use_tc_tiling_on_sc=False)
    return cp


_MESH = plsc.VectorSubcoreMesh(core_axis_name="c", subcore_axis_name="s")


def _sc_spmm(rows, cols, vals, weight):
    nnz = rows.shape[0]
    cap = NS * (NSB0 + NSB1) * SB
    assert cap >= nnz, (cap, nnz)
    pad = cap - nnz
    if pad:
        rows = jnp.concatenate([rows, jnp.zeros((pad,), rows.dtype)])
        cols = jnp.concatenate([cols, jnp.zeros((pad,), cols.dtype)])
        vals = jnp.concatenate([vals, jnp.zeros((pad,), vals.dtype)])
    rows = rows.reshape(-1, Q)

    @functools.partial(
        pl.kernel,
        mesh=_MESH,
        compiler_params=_mk_cp(),
        out_type=jax.ShapeDtypeStruct((NC, N, D), jnp.float32),
        scratch_types=[
            pltpu.VMEM((EPWM // Q, Q), jnp.int32),
            pltpu.VMEM((NCB, SB), jnp.int32),
            pltpu.VMEM((NGB, SB), jnp.float32),
            pltpu.VMEM((NGB, SB, D), jnp.float32),
            pltpu.VMEM_SHARED((N, D), jnp.float32),
            pltpu.SemaphoreType.DMA,
            pltpu.SemaphoreType.DMA((NCB,)),
            pltpu.SemaphoreType.DMA((NGB,)),
            pltpu.SemaphoreType.DMA((NGB,)),
            pltpu.SemaphoreType.DMA((NGB,)),
        ],
    )
    def spmm(rows_hbm, cols_hbm, vals_hbm, w_hbm, part_hbm,
             rows_v, cols_v, vals_v, g_v, acc, sem_i, sem_c, sem_v,
             sem_g, sem_s):
        cid = lax.axis_index("c")
        sid = lax.axis_index("s")
        my_nsb = jnp.where(cid == 0, NSB0, NSB1)
        base = jnp.where(cid == 0, sid * NSB0, NS * NSB0 + sid * NSB1) * SB

        idx_c = [
            pltpu.async_copy(rows_hbm.at[pl.ds(base // Q, EPWM // Q)],
                             rows_v, sem_i),
        ]

        def issue_cols(s):
            b = s % NCB
            return pltpu.async_copy(cols_hbm.at[pl.ds(base + s * SB, SB)],
                                    cols_v.at[b], sem_c.at[b])

        def wait_cols(s):
            b = s % NCB
            pltpu.make_async_copy(cols_hbm.at[pl.ds(0, SB)], cols_v.at[b],
                                  sem_c.at[b]).wait()

        def issue_vals(s):
            b = s % NGB
            return pltpu.async_copy(vals_hbm.at[pl.ds(base + s * SB, SB)],
                                    vals_v.at[b], sem_v.at[b])

        def wait_vals(s):
            b = s % NGB
            pltpu.make_async_copy(vals_hbm.at[pl.ds(0, SB)], vals_v.at[b],
                                  sem_v.at[b]).wait()

        @pl.loop(0, SB)
        def _(r):
            for j in range(D // 16):
                g_v[0, r, pl.ds(j * 16, 16)] = jnp.zeros((16,), jnp.float32)

        zbase = sid * ROWS_PER_TILE
        done = 0
        while done < ROWS_PER_TILE:
            step = min(SB, ROWS_PER_TILE - done)
            pltpu.sync_copy(g_v.at[0, pl.ds(0, step)],
                            acc.at[pl.ds(zbase + done, step)])
            done += step
        plsc.subcore_barrier()

        for c in idx_c:
            c.wait()

        def issue_gather(s):
            b = s % NGB
            wait_cols(s)
            return pltpu.async_copy(w_hbm.at[cols_v.at[s % NCB]],
                                    g_v.at[b], sem_g.at[b])

        def guarded(s, fn):
            def fn_none():
                fn()

            if s < min(NSB0, NSB1):
                fn_none()
            else:
                pl.when(s < my_nsb)(fn_none)

        issue_cols(0)
        for t in range(1, min(3, NSBM)):
            guarded(t, lambda: issue_cols(t))
        issue_vals(0)
        issue_gather(0)
        if NSBM > 1:
            guarded(1, lambda: issue_vals(1))
        s_c = {}

        for s in range(NSBM):
            b = s % NGB
            if s - 2 >= 0:
                cs = s_c.pop(s - 2)
                guarded(s - 2, lambda: [c.wait() for c in cs])
            if s + 1 < NSBM:
                guarded(s + 1, lambda: issue_gather(s + 1))

            def work():
                pltpu.make_async_copy(
                    w_hbm.at[pl.ds(0, SB)], g_v.at[b],
                    sem_g.at[b]).wait()
                wait_vals(s)
                vref = vals_v.at[b]

                @pl.loop(0, SB, step=4)
                def _(e0):
                    for u in range(4):
                        e = e0 + u
                        vb = plsc.load_gather(
                            vref, [jnp.full((16,), e, jnp.int32)])
                        for j in range(D // 16):
                            g_v[b, e, pl.ds(j * 16, 16)] = (
                                g_v[b, e, pl.ds(j * 16, 16)] * vb)

                s_c[s] = [
                    pltpu.async_copy(g_v.at[b, pl.ds(q * Q, Q)],
                                     acc.at[rows_v.at[NQ * s + q]],
                                     sem_s.at[b], add=True)
                    for q in range(NQ)
                ]

            guarded(s, work)
            if s not in s_c:
                s_c[s] = []
            if s + 2 < NSBM:
                guarded(s + 2, lambda: issue_vals(s + 2))
            if s + 3 < NSBM:
                guarded(s + 3, lambda: issue_cols(s + 3))

        for t in range(max(NSBM - 2, 0), NSBM):
            cs = s_c.pop(t)
            guarded(t, lambda: [c.wait() for c in cs])

        plsc.subcore_barrier()
        pltpu.sync_copy(
            acc.at[pl.ds(sid * ROWS_PER_TILE, ROWS_PER_TILE)],
            part_hbm.at[cid, pl.ds(sid * ROWS_PER_TILE, ROWS_PER_TILE)],
        )

    return spmm(rows, cols, vals, weight)


def _sc_combine(part):
    RW = N // NW
    RH = RW // 4

    @functools.partial(
        pl.kernel,
        mesh=_MESH,
        compiler_params=_mk_cp(),
        out_type=jax.ShapeDtypeStruct((N, D), jnp.float32),
        scratch_types=[
            pltpu.VMEM((2, RH, D), jnp.float32),
            pltpu.VMEM((2, RH, D), jnp.float32),
            pltpu.SemaphoreType.DMA((2,)),
            pltpu.SemaphoreType.DMA((2,)),
        ],
    )
    def comb(part_hbm, out_hbm, a_v, b_v, sem_a, sem_b):
        cid = lax.axis_index("c")
        sid = lax.axis_index("s")
        r0 = (sid * NC + cid) * RW
        def issue(h):
            b = h % 2
            return [
                pltpu.async_copy(part_hbm.at[0, pl.ds(r0 + h * RH, RH)],
                                 a_v.at[b], sem_a.at[b]),
                pltpu.async_copy(part_hbm.at[1, pl.ds(r0 + h * RH, RH)],
                                 b_v.at[b], sem_b.at[b]),
            ]

        cps = {0: issue(0)}
        for h in range(RW // RH):
            b = h % 2
            if h + 1 < RW // RH:
                cps[h + 1] = issue(h + 1)
            for c in cps.pop(h):
                c.wait()

            @pl.loop(0, RH)
            def _(r):
                for j in range(D // 16):
                    a_v[b, r, pl.ds(j * 16, 16)] = (
                        a_v[b, r, pl.ds(j * 16, 16)]
                        + b_v[b, r, pl.ds(j * 16, 16)])

            pltpu.sync_copy(a_v.at[b], out_hbm.at[pl.ds(r0 + h * RH, RH)])

    return comb(part)


def kernel(rows, cols, vals, weight):
    rows = rows.astype(jnp.int32)
    cols = cols.astype(jnp.int32)
    part = _sc_spmm(rows, cols, vals, weight)
    return _sc_combine(part)
